# baseline (speedup 1.0000x reference)
import functools

import jax
import jax.numpy as jnp
from jax import lax
from jax.experimental import pallas as pl
from jax.experimental.pallas import tpu as pltpu
from jax.experimental.pallas import tpu_sc as plsc

V = 1000
D = 1000
B = 1024
T = 200
NW = 32
TB = T // 8
DB = D // 8
NP = TB * DB
L = 16

_mesh = plsc.VectorSubcoreMesh(core_axis_name="c", subcore_axis_name="s")


@functools.partial(
    pl.kernel,
    mesh=_mesh,
    compiler_params=pltpu.CompilerParams(
        use_tc_tiling_on_sc=True, needs_layout_passes=False),
    out_type=jax.ShapeDtypeStruct((T, D, B), jnp.float32),
    scratch_types=[
        pltpu.VMEM((8, B), jnp.int32),
        pltpu.VMEM((8, B), jnp.int32),
        pltpu.VMEM((64, 128), jnp.float32),
        pltpu.VMEM((64, 128), jnp.float32),
        pltpu.VMEM((4, 8, B), jnp.float32),
        pltpu.VMEM((4, 8, B), jnp.float32),
        pltpu.SemaphoreType.DMA,
        pltpu.SemaphoreType.DMA,
        pltpu.SemaphoreType.DMA,
        pltpu.SemaphoreType.DMA,
    ],
)
def _gather(idxt_hbm, tab_hbm, out_hbm,
            i0, i1, s0, s1, o0, o1, sl0, sl1, so0, so1):
    sid = lax.axis_index("s")
    wid = sid * 2 + lax.axis_index("c")

    ib, sb, ob = (i0, i1), (s0, s1), (o0, o1)
    sl, so = (sl0, sl1), (so0, so1)

    KMAX = (NP + NW - 1) // NW

    def pair(k):
        p = lax.min(wid + k * NW, NP - 1)
        return lax.div(p, DB), lax.rem(p, DB)

    def start_loads(k, nb):
        tb, db = pair(k)
        pltpu.make_async_copy(
            idxt_hbm.at[pl.ds(tb * 8, 8)], ib[nb], sl[nb]
        ).start()
        pltpu.make_async_copy(tab_hbm.at[db], sb[nb], sl[nb]).start()

    def wait_loads(nb):
        pltpu.make_async_copy(
            idxt_hbm.at[pl.ds(0, 8)], ib[nb], sl[nb]
        ).wait()
        pltpu.make_async_copy(tab_hbm.at[0], sb[nb], sl[nb]).wait()

    def wait_out(nb):
        pltpu.make_async_copy(
            ob[nb], out_hbm.at[pl.ds(0, 4), pl.ds(0, 8)], so[nb]
        ).wait()

    start_loads(0, 0)

    def body(k2, carry):
        for kb in range(2):
            k = k2 * 2 + kb
            tb, db = pair(k)
            wait_loads(kb)
            start_loads(k + 1, 1 - kb)

            for th in range(2):
                @pl.when(k >= 1)
                def _():
                    wait_out(th)

                for tt in range(4):
                    def jbody(j, c):
                        bidx = ib[kb][th * 4 + tt, pl.ds(j * L, L)]
                        row0 = (bidx >> 7) * 8
                        col = bidx & 127
                        for d_ in range(8):
                            vals = plsc.load_gather(
                                sb[kb], [row0 + d_, col])
                            ob[th][tt, d_, pl.ds(j * L, L)] = vals
                        return c

                    lax.fori_loop(0, B // L, jbody, 0)

                pltpu.make_async_copy(
                    ob[th],
                    out_hbm.at[pl.ds(tb * 8 + th * 4, 4),
                               pl.ds(db * 8, 8)],
                    so[th],
                ).start()
        return carry

    lax.fori_loop(0, KMAX // 2, body, 0)

    wait_loads(KMAX % 2)
    for th in range(2):
        wait_out(th)


def kernel(idx, table):
    idxt = jnp.swapaxes(idx, 0, 1)
    tt = jnp.pad(table.T, ((0, 0), (0, 24)))
    t4 = tt.reshape(DB, 8, 8, 128).transpose(0, 2, 1, 3).reshape(DB, 64, 128)
    out = _gather(idxt, t4)
    return jnp.transpose(out, (2, 0, 1))


# speedup vs baseline: 3.3778x; 3.3778x over previous
import functools

import jax
import jax.numpy as jnp
from jax import lax
from jax.experimental import pallas as pl
from jax.experimental.pallas import tpu as pltpu
from jax.experimental.pallas import tpu_sc as plsc

V = 1000
D = 1000
B = 1024
T = 200
NW = 32
TB = T // 8
DB = D // 8
NP = TB * DB
L = 16

_mesh = plsc.VectorSubcoreMesh(core_axis_name="c", subcore_axis_name="s")


@functools.partial(
    pl.kernel,
    mesh=_mesh,
    compiler_params=pltpu.CompilerParams(
        use_tc_tiling_on_sc=True, needs_layout_passes=False),
    out_type=jax.ShapeDtypeStruct((T, D, B), jnp.float32),
    scratch_types=[
        pltpu.VMEM((8, B), jnp.int32),
        pltpu.VMEM((8, B), jnp.int32),
        pltpu.VMEM((64, 128), jnp.float32),
        pltpu.VMEM((64, 128), jnp.float32),
        pltpu.VMEM((4, 8, B), jnp.float32),
        pltpu.VMEM((4, 8, B), jnp.float32),
        pltpu.SemaphoreType.DMA,
        pltpu.SemaphoreType.DMA,
        pltpu.SemaphoreType.DMA,
        pltpu.SemaphoreType.DMA,
    ],
)
def _gather(idxt_hbm, tab_hbm, out_hbm,
            i0, i1, s0, s1, o0, o1, sl0, sl1, so0, so1):
    sid = lax.axis_index("s")
    wid = sid * 2 + lax.axis_index("c")

    ib, sb, ob = (i0, i1), (s0, s1), (o0, o1)
    sl, so = (sl0, sl1), (so0, so1)

    KMAX = (NP + NW - 1) // NW

    def pair(k):
        p = lax.min(wid + k * NW, NP - 1)
        return lax.div(p, DB), lax.rem(p, DB)

    def start_loads(k, nb):
        tb, db = pair(k)
        pltpu.make_async_copy(
            idxt_hbm.at[pl.ds(tb * 8, 8)], ib[nb], sl[nb]
        ).start()
        pltpu.make_async_copy(tab_hbm.at[db], sb[nb], sl[nb]).start()

    def wait_loads(nb):
        pltpu.make_async_copy(
            idxt_hbm.at[pl.ds(0, 8)], ib[nb], sl[nb]
        ).wait()
        pltpu.make_async_copy(tab_hbm.at[0], sb[nb], sl[nb]).wait()

    def wait_out(nb):
        pltpu.make_async_copy(
            ob[nb], out_hbm.at[pl.ds(0, 4), pl.ds(0, 8)], so[nb]
        ).wait()

    start_loads(0, 0)

    def body(k2, carry):
        for kb in range(2):
            k = k2 * 2 + kb
            tb, db = pair(k)
            wait_loads(kb)
            start_loads(k + 1, 1 - kb)

            for th in range(2):
                @pl.when(k >= 1)
                def _():
                    wait_out(th)

                for tt in range(4):
                    @plsc.parallel_loop(0, B // L, 1, unroll=4)
                    def jbody(j):
                        bidx = ib[kb][th * 4 + tt, pl.ds(j * L, L)]
                        row0 = (bidx >> 7) * 8
                        col = bidx & 127
                        for d_ in range(8):
                            vals = plsc.load_gather(
                                sb[kb], [row0 + d_, col])
                            ob[th][tt, d_, pl.ds(j * L, L)] = vals

                pltpu.make_async_copy(
                    ob[th],
                    out_hbm.at[pl.ds(tb * 8 + th * 4, 4),
                               pl.ds(db * 8, 8)],
                    so[th],
                ).start()
        return carry

    lax.fori_loop(0, KMAX // 2, body, 0)

    wait_loads(KMAX % 2)
    for th in range(2):
        wait_out(th)


def kernel(idx, table):
    idxt = jnp.swapaxes(idx, 0, 1)
    tt = jnp.pad(table.T, ((0, 0), (0, 24)))
    t4 = tt.reshape(DB, 8, 8, 128).transpose(0, 2, 1, 3).reshape(DB, 64, 128)
    out = _gather(idxt, t4)
    return jnp.transpose(out, (2, 0, 1))
